# baseline (speedup 1.0000x reference)
import functools

import jax
import jax.numpy as jnp
from jax.experimental import pallas as pl

S = 256
K = 26
H = 256
STEPS = 3
GATE_PAD = 128


def _body(cur_ref, nb_ref, wg_ref, bg_ref, wl_ref, bl_ref, w1_ref, b1_ref,
          w2_ref, b2_ref, wo1_ref, bo1_ref, wo2_ref, bo2_ref, out_ref):
    f32 = jnp.float32
    x = cur_ref[...]

    w1c = w1_ref[:S, :]
    w1n = w1_ref[S:, :]
    cur_msg = jnp.dot(x, w1c, preferred_element_type=f32) + b1_ref[...]
    nb_sum = jnp.zeros((x.shape[0], S), f32)
    msg_acc = jnp.zeros((x.shape[0], H), f32)
    for k in range(K):
        nk = nb_ref[:, k * S:(k + 1) * S]
        nb_sum = nb_sum + nk
        m = jnp.dot(nk, w1n, preferred_element_type=f32) + cur_msg
        msg_acc = msg_acc + jnp.maximum(m, 0.0)
    nb_mean = nb_sum * (1.0 / K)
    agg = msg_acc * (1.0 / K)

    logits = (jnp.dot(x, wg_ref[:S, :], preferred_element_type=f32)
              + jnp.dot(nb_mean, wg_ref[S:, :], preferred_element_type=f32)
              + bg_ref[...])
    l0 = logits[:, 0:1]
    l1 = logits[:, 1:2]
    l2 = logits[:, 2:3]
    mx = jnp.maximum(jnp.maximum(l0, l1), l2)
    e0 = jnp.exp(l0 - mx)
    e1 = jnp.exp(l1 - mx)
    e2 = jnp.exp(l2 - mx)
    inv = 1.0 / (e0 + e1 + e2)

    local = jnp.tanh(jnp.dot(x, wl_ref[:S, :], preferred_element_type=f32)
                     + jnp.dot(nb_mean, wl_ref[S:, :], preferred_element_type=f32)
                     + bl_ref[...])

    func = jnp.tanh(jnp.dot(agg, w2_ref[:H, :], preferred_element_type=f32)
                    + jnp.dot(x, w2_ref[H:, :], preferred_element_type=f32)
                    + b2_ref[...])

    dt = 1.0 / STEPS
    state = x
    for _ in range(STEPS):
        h = jnp.tanh(jnp.dot(state, wo1_ref[...], preferred_element_type=f32)
                     + bo1_ref[...])
        dstate = jnp.dot(h, wo2_ref[...], preferred_element_type=f32) + bo2_ref[...]
        state = state + dt * dstate

    out_ref[...] = (e0 * local + e1 * func + e2 * state) * inv


@functools.partial(jax.jit, static_argnames=())
def _run(current_state, nb2d, wg_pad, bg_pad, W_local, b_local, W_msg1,
         b_msg1, W_msg2, b_msg2, W_ode1, b_ode1, W_ode2, b_ode2):
    B = current_state.shape[0]
    bB = 512
    grid = (B // bB,)

    def row_map(i):
        return (i, 0)

    def fixed_map(i):
        return (0, 0)

    in_specs = [
        pl.BlockSpec((bB, S), row_map),
        pl.BlockSpec((bB, K * S), row_map),
        pl.BlockSpec((2 * S, GATE_PAD), fixed_map),
        pl.BlockSpec((1, GATE_PAD), fixed_map),
        pl.BlockSpec((2 * S, S), fixed_map),
        pl.BlockSpec((1, S), fixed_map),
        pl.BlockSpec((2 * S, H), fixed_map),
        pl.BlockSpec((1, H), fixed_map),
        pl.BlockSpec((H + S, S), fixed_map),
        pl.BlockSpec((1, S), fixed_map),
        pl.BlockSpec((S, H), fixed_map),
        pl.BlockSpec((1, H), fixed_map),
        pl.BlockSpec((H, S), fixed_map),
        pl.BlockSpec((1, S), fixed_map),
    ]
    return pl.pallas_call(
        _body,
        grid=grid,
        in_specs=in_specs,
        out_specs=pl.BlockSpec((bB, S), row_map),
        out_shape=jax.ShapeDtypeStruct((B, S), jnp.float32),
    )(current_state, nb2d, wg_pad, bg_pad, W_local, b_local.reshape(1, S),
      W_msg1, b_msg1.reshape(1, H), W_msg2, b_msg2.reshape(1, S),
      W_ode1, b_ode1.reshape(1, H), W_ode2, b_ode2.reshape(1, S))


def kernel(current_state, neighbor_states, W_gate, b_gate, W_local, b_local,
           W_msg1, b_msg1, W_msg2, b_msg2, W_ode1, b_ode1, W_ode2, b_ode2,
           cell_idx, neighbor_indices):
    B = current_state.shape[0]
    nb2d = neighbor_states.reshape(B, K * S)
    wg_pad = jnp.zeros((2 * S, GATE_PAD), jnp.float32).at[:, :3].set(W_gate)
    bg_pad = jnp.zeros((1, GATE_PAD), jnp.float32).at[0, :3].set(b_gate)
    return _run(current_state, nb2d, wg_pad, bg_pad, W_local, b_local,
                W_msg1, b_msg1, W_msg2, b_msg2, W_ode1, b_ode1, W_ode2,
                b_ode2)


# speedup vs baseline: 1.3997x; 1.3997x over previous
import functools

import jax
import jax.numpy as jnp
from jax.experimental import pallas as pl

S = 256
K = 26
H = 256
STEPS = 3
GATE_PAD = 128
KG = K // 8
KR = K - 8 * KG


def _body(cur_ref, nb_ref, wg_ref, bg_ref, wl_ref, bl_ref, w1_ref, b1_ref,
          w2_ref, b2_ref, wo1_ref, bo1_ref, wo2_ref, bo2_ref, out_ref):
    f32 = jnp.float32
    x = cur_ref[...]
    bB = x.shape[0]

    w1c = w1_ref[:S, :]
    w1n = w1_ref[S:, :]
    cur_msg = jnp.dot(x, w1c, preferred_element_type=f32) + b1_ref[...]
    cur8 = jnp.broadcast_to(cur_msg[:, None, :], (bB, 8, H)).reshape(bB * 8, H)
    nacc = jnp.zeros((bB * 8, S), f32)
    macc = jnp.zeros((bB * 8, H), f32)
    for g in range(KG):
        flat = nb_ref[:, 8 * g:8 * g + 8, :].reshape(bB * 8, S)
        nacc = nacc + flat
        m = jnp.dot(flat, w1n, preferred_element_type=f32) + cur8
        macc = macc + jnp.maximum(m, 0.0)
    nb_sum = jnp.sum(nacc.reshape(bB, 8, S), axis=1)
    msum = jnp.sum(macc.reshape(bB, 8, H), axis=1)
    for k in range(8 * KG, K):
        nk = nb_ref[:, k, :]
        nb_sum = nb_sum + nk
        m = jnp.dot(nk, w1n, preferred_element_type=f32) + cur_msg
        msum = msum + jnp.maximum(m, 0.0)
    nb_mean = nb_sum * (1.0 / K)
    agg = msum * (1.0 / K)

    logits = (jnp.dot(x, wg_ref[:S, :], preferred_element_type=f32)
              + jnp.dot(nb_mean, wg_ref[S:, :], preferred_element_type=f32)
              + bg_ref[...])
    l0 = logits[:, 0:1]
    l1 = logits[:, 1:2]
    l2 = logits[:, 2:3]
    mx = jnp.maximum(jnp.maximum(l0, l1), l2)
    e0 = jnp.exp(l0 - mx)
    e1 = jnp.exp(l1 - mx)
    e2 = jnp.exp(l2 - mx)
    inv = 1.0 / (e0 + e1 + e2)

    local = jnp.tanh(jnp.dot(x, wl_ref[:S, :], preferred_element_type=f32)
                     + jnp.dot(nb_mean, wl_ref[S:, :], preferred_element_type=f32)
                     + bl_ref[...])

    func = jnp.tanh(jnp.dot(agg, w2_ref[:H, :], preferred_element_type=f32)
                    + jnp.dot(x, w2_ref[H:, :], preferred_element_type=f32)
                    + b2_ref[...])

    dt = 1.0 / STEPS
    state = x
    for _ in range(STEPS):
        h = jnp.tanh(jnp.dot(state, wo1_ref[...], preferred_element_type=f32)
                     + bo1_ref[...])
        dstate = jnp.dot(h, wo2_ref[...], preferred_element_type=f32) + bo2_ref[...]
        state = state + dt * dstate

    out_ref[...] = (e0 * local + e1 * func + e2 * state) * inv


@jax.jit
def _run(current_state, neighbor_states, wg_pad, bg_pad, W_local, b_local,
         W_msg1, b_msg1, W_msg2, b_msg2, W_ode1, b_ode1, W_ode2, b_ode2):
    B = current_state.shape[0]
    bB = 512
    grid = (B // bB,)

    def row_map(i):
        return (i, 0)

    def row_map3(i):
        return (i, 0, 0)

    def fixed_map(i):
        return (0, 0)

    in_specs = [
        pl.BlockSpec((bB, S), row_map),
        pl.BlockSpec((bB, K, S), row_map3),
        pl.BlockSpec((2 * S, GATE_PAD), fixed_map),
        pl.BlockSpec((1, GATE_PAD), fixed_map),
        pl.BlockSpec((2 * S, S), fixed_map),
        pl.BlockSpec((1, S), fixed_map),
        pl.BlockSpec((2 * S, H), fixed_map),
        pl.BlockSpec((1, H), fixed_map),
        pl.BlockSpec((H + S, S), fixed_map),
        pl.BlockSpec((1, S), fixed_map),
        pl.BlockSpec((S, H), fixed_map),
        pl.BlockSpec((1, H), fixed_map),
        pl.BlockSpec((H, S), fixed_map),
        pl.BlockSpec((1, S), fixed_map),
    ]
    return pl.pallas_call(
        _body,
        grid=grid,
        in_specs=in_specs,
        out_specs=pl.BlockSpec((bB, S), row_map),
        out_shape=jax.ShapeDtypeStruct((B, S), jnp.float32),
    )(current_state, neighbor_states, wg_pad, bg_pad, W_local,
      b_local.reshape(1, S), W_msg1, b_msg1.reshape(1, H), W_msg2,
      b_msg2.reshape(1, S), W_ode1, b_ode1.reshape(1, H), W_ode2,
      b_ode2.reshape(1, S))


def kernel(current_state, neighbor_states, W_gate, b_gate, W_local, b_local,
           W_msg1, b_msg1, W_msg2, b_msg2, W_ode1, b_ode1, W_ode2, b_ode2,
           cell_idx, neighbor_indices):
    wg_pad = jnp.zeros((2 * S, GATE_PAD), jnp.float32).at[:, :3].set(W_gate)
    bg_pad = jnp.zeros((1, GATE_PAD), jnp.float32).at[0, :3].set(b_gate)
    return _run(current_state, neighbor_states, wg_pad, bg_pad, W_local,
                b_local, W_msg1, b_msg1, W_msg2, b_msg2, W_ode1, b_ode1,
                W_ode2, b_ode2)


# speedup vs baseline: 1.4058x; 1.0043x over previous
import functools

import jax
import jax.numpy as jnp
from jax.experimental import pallas as pl

S = 256
K = 26
H = 256
STEPS = 3
GATE_PAD = 128
KG = K // 8
KR = K - 8 * KG


def _nb_accumulate(nb_ref, cur_msg, w1n):
    f32 = jnp.float32
    bB = nb_ref.shape[0]
    cur8 = jnp.broadcast_to(cur_msg[:, None, :], (bB, 8, H)).reshape(bB * 8, H)
    nacc = jnp.zeros((bB * 8, S), f32)
    macc = jnp.zeros((bB * 8, H), f32)
    for g in range(KG):
        flat = nb_ref[:, 8 * g:8 * g + 8, :].reshape(bB * 8, S)
        nacc = nacc + flat
        m = jnp.dot(flat, w1n, preferred_element_type=f32) + cur8
        macc = macc + jnp.maximum(m, 0.0)
    nb_sum = jnp.sum(nacc.reshape(bB, 8, S), axis=1)
    msum = jnp.sum(macc.reshape(bB, 8, H), axis=1)
    for k in range(8 * KG, K):
        nk = nb_ref[:, k, :]
        nb_sum = nb_sum + nk
        m = jnp.dot(nk, w1n, preferred_element_type=f32) + cur_msg
        msum = msum + jnp.maximum(m, 0.0)
    return nb_sum, msum


def _body(cur_ref, nb_ref_a, nb_ref_b, wg_ref, bg_ref, wl_ref, bl_ref,
          w1_ref, b1_ref, w2_ref, b2_ref, wo1_ref, bo1_ref, wo2_ref, bo2_ref,
          out_ref):
    f32 = jnp.float32
    x = cur_ref[...]
    hB = nb_ref_a.shape[0]

    w1c = w1_ref[:S, :]
    w1n = w1_ref[S:, :]
    cur_msg = jnp.dot(x, w1c, preferred_element_type=f32) + b1_ref[...]
    nb_sum_a, msum_a = _nb_accumulate(nb_ref_a, cur_msg[:hB], w1n)
    nb_sum_b, msum_b = _nb_accumulate(nb_ref_b, cur_msg[hB:], w1n)
    nb_sum = jnp.concatenate([nb_sum_a, nb_sum_b], axis=0)
    msum = jnp.concatenate([msum_a, msum_b], axis=0)
    nb_mean = nb_sum * (1.0 / K)
    agg = msum * (1.0 / K)

    logits = (jnp.dot(x, wg_ref[:S, :], preferred_element_type=f32)
              + jnp.dot(nb_mean, wg_ref[S:, :], preferred_element_type=f32)
              + bg_ref[...])
    l0 = logits[:, 0:1]
    l1 = logits[:, 1:2]
    l2 = logits[:, 2:3]
    mx = jnp.maximum(jnp.maximum(l0, l1), l2)
    e0 = jnp.exp(l0 - mx)
    e1 = jnp.exp(l1 - mx)
    e2 = jnp.exp(l2 - mx)
    inv = 1.0 / (e0 + e1 + e2)

    local = jnp.tanh(jnp.dot(x, wl_ref[:S, :], preferred_element_type=f32)
                     + jnp.dot(nb_mean, wl_ref[S:, :], preferred_element_type=f32)
                     + bl_ref[...])

    func = jnp.tanh(jnp.dot(agg, w2_ref[:H, :], preferred_element_type=f32)
                    + jnp.dot(x, w2_ref[H:, :], preferred_element_type=f32)
                    + b2_ref[...])

    dt = 1.0 / STEPS
    state = x
    for _ in range(STEPS):
        h = jnp.tanh(jnp.dot(state, wo1_ref[...], preferred_element_type=f32)
                     + bo1_ref[...])
        dstate = jnp.dot(h, wo2_ref[...], preferred_element_type=f32) + bo2_ref[...]
        state = state + dt * dstate

    out_ref[...] = (e0 * local + e1 * func + e2 * state) * inv


@jax.jit
def _run(current_state, neighbor_states, wg_pad, bg_pad, W_local, b_local,
         W_msg1, b_msg1, W_msg2, b_msg2, W_ode1, b_ode1, W_ode2, b_ode2):
    B = current_state.shape[0]
    bB = 512
    grid = (B // bB,)

    def row_map(i):
        return (i, 0)

    def half_map_a(i):
        return (2 * i, 0, 0)

    def half_map_b(i):
        return (2 * i + 1, 0, 0)

    def fixed_map(i):
        return (0, 0)

    in_specs = [
        pl.BlockSpec((bB, S), row_map),
        pl.BlockSpec((bB // 2, K, S), half_map_a),
        pl.BlockSpec((bB // 2, K, S), half_map_b),
        pl.BlockSpec((2 * S, GATE_PAD), fixed_map),
        pl.BlockSpec((1, GATE_PAD), fixed_map),
        pl.BlockSpec((2 * S, S), fixed_map),
        pl.BlockSpec((1, S), fixed_map),
        pl.BlockSpec((2 * S, H), fixed_map),
        pl.BlockSpec((1, H), fixed_map),
        pl.BlockSpec((H + S, S), fixed_map),
        pl.BlockSpec((1, S), fixed_map),
        pl.BlockSpec((S, H), fixed_map),
        pl.BlockSpec((1, H), fixed_map),
        pl.BlockSpec((H, S), fixed_map),
        pl.BlockSpec((1, S), fixed_map),
    ]
    return pl.pallas_call(
        _body,
        grid=grid,
        in_specs=in_specs,
        out_specs=pl.BlockSpec((bB, S), row_map),
        out_shape=jax.ShapeDtypeStruct((B, S), jnp.float32),
    )(current_state, neighbor_states, neighbor_states, wg_pad, bg_pad, W_local,
      b_local.reshape(1, S), W_msg1, b_msg1.reshape(1, H), W_msg2,
      b_msg2.reshape(1, S), W_ode1, b_ode1.reshape(1, H), W_ode2,
      b_ode2.reshape(1, S))


def kernel(current_state, neighbor_states, W_gate, b_gate, W_local, b_local,
           W_msg1, b_msg1, W_msg2, b_msg2, W_ode1, b_ode1, W_ode2, b_ode2,
           cell_idx, neighbor_indices):
    wg_pad = jnp.zeros((2 * S, GATE_PAD), jnp.float32).at[:, :3].set(W_gate)
    bg_pad = jnp.zeros((1, GATE_PAD), jnp.float32).at[0, :3].set(b_gate)
    return _run(current_state, neighbor_states, wg_pad, bg_pad, W_local,
                b_local, W_msg1, b_msg1, W_msg2, b_msg2, W_ode1, b_ode1,
                W_ode2, b_ode2)


# speedup vs baseline: 4.5929x; 3.2672x over previous
import jax
import jax.numpy as jnp
from jax.experimental import pallas as pl

S = 256
K = 26
H = 256
STEPS = 3
GATE_PAD = 128


def _body(cur_ref, nb_ref, wg_ref, bg_ref, wl_ref, bl_ref, w1_ref, b1_ref,
          w2_ref, b2_ref, wo1_ref, bo1_ref, wo2_ref, bo2_ref, out_ref):
    f32 = jnp.float32
    bf16 = jnp.bfloat16
    x = cur_ref[...]
    xb = x.astype(bf16)

    w1c = w1_ref[:S, :].astype(bf16)
    w1n = w1_ref[S:, :].astype(bf16)
    cur_msg = jnp.dot(xb, w1c, preferred_element_type=f32) + b1_ref[...]
    nb_sum = jnp.zeros_like(x)
    msum = jnp.zeros((x.shape[0], H), f32)
    for k in range(K):
        nk = nb_ref[k]
        nb_sum = nb_sum + nk
        m = jnp.dot(nk.astype(bf16), w1n, preferred_element_type=f32) + cur_msg
        msum = msum + jnp.maximum(m, 0.0)
    nb_mean = nb_sum * (1.0 / K)
    nb_mean_b = nb_mean.astype(bf16)
    agg = msum * (1.0 / K)

    logits = (jnp.dot(xb, wg_ref[:S, :].astype(bf16), preferred_element_type=f32)
              + jnp.dot(nb_mean_b, wg_ref[S:, :].astype(bf16),
                        preferred_element_type=f32)
              + bg_ref[...])
    l0 = logits[:, 0:1]
    l1 = logits[:, 1:2]
    l2 = logits[:, 2:3]
    mx = jnp.maximum(jnp.maximum(l0, l1), l2)
    e0 = jnp.exp(l0 - mx)
    e1 = jnp.exp(l1 - mx)
    e2 = jnp.exp(l2 - mx)
    inv = 1.0 / (e0 + e1 + e2)

    local = jnp.tanh(
        jnp.dot(xb, wl_ref[:S, :].astype(bf16), preferred_element_type=f32)
        + jnp.dot(nb_mean_b, wl_ref[S:, :].astype(bf16),
                  preferred_element_type=f32)
        + bl_ref[...])

    func = jnp.tanh(
        jnp.dot(agg.astype(bf16), w2_ref[:H, :].astype(bf16),
                preferred_element_type=f32)
        + jnp.dot(xb, w2_ref[H:, :].astype(bf16), preferred_element_type=f32)
        + b2_ref[...])

    dt = 1.0 / STEPS
    wo1 = wo1_ref[...].astype(bf16)
    wo2 = wo2_ref[...].astype(bf16)
    state = x
    for _ in range(STEPS):
        h = jnp.tanh(jnp.dot(state.astype(bf16), wo1,
                             preferred_element_type=f32) + bo1_ref[...])
        dstate = jnp.dot(h.astype(bf16), wo2,
                         preferred_element_type=f32) + bo2_ref[...]
        state = state + dt * dstate

    out_ref[...] = (e0 * local + e1 * func + e2 * state) * inv


@jax.jit
def _run(current_state, nb_t, wg_pad, bg_pad, W_local, b_local,
         W_msg1, b_msg1, W_msg2, b_msg2, W_ode1, b_ode1, W_ode2, b_ode2):
    B = current_state.shape[0]
    bB = 512
    grid = (B // bB,)

    def row_map(i):
        return (i, 0)

    def nb_map(i):
        return (0, i, 0)

    def fixed_map(i):
        return (0, 0)

    in_specs = [
        pl.BlockSpec((bB, S), row_map),
        pl.BlockSpec((K, bB, S), nb_map),
        pl.BlockSpec((2 * S, GATE_PAD), fixed_map),
        pl.BlockSpec((1, GATE_PAD), fixed_map),
        pl.BlockSpec((2 * S, S), fixed_map),
        pl.BlockSpec((1, S), fixed_map),
        pl.BlockSpec((2 * S, H), fixed_map),
        pl.BlockSpec((1, H), fixed_map),
        pl.BlockSpec((H + S, S), fixed_map),
        pl.BlockSpec((1, S), fixed_map),
        pl.BlockSpec((S, H), fixed_map),
        pl.BlockSpec((1, H), fixed_map),
        pl.BlockSpec((H, S), fixed_map),
        pl.BlockSpec((1, S), fixed_map),
    ]
    return pl.pallas_call(
        _body,
        grid=grid,
        in_specs=in_specs,
        out_specs=pl.BlockSpec((bB, S), row_map),
        out_shape=jax.ShapeDtypeStruct((B, S), jnp.float32),
    )(current_state, nb_t, wg_pad, bg_pad, W_local,
      b_local.reshape(1, S), W_msg1, b_msg1.reshape(1, H), W_msg2,
      b_msg2.reshape(1, S), W_ode1, b_ode1.reshape(1, H), W_ode2,
      b_ode2.reshape(1, S))


def kernel(current_state, neighbor_states, W_gate, b_gate, W_local, b_local,
           W_msg1, b_msg1, W_msg2, b_msg2, W_ode1, b_ode1, W_ode2, b_ode2,
           cell_idx, neighbor_indices):
    nb_t = jnp.transpose(neighbor_states, (1, 0, 2))
    wg_pad = jnp.zeros((2 * S, GATE_PAD), jnp.float32).at[:, :3].set(W_gate)
    bg_pad = jnp.zeros((1, GATE_PAD), jnp.float32).at[0, :3].set(b_gate)
    return _run(current_state, nb_t, wg_pad, bg_pad, W_local,
                b_local, W_msg1, b_msg1, W_msg2, b_msg2, W_ode1, b_ode1,
                W_ode2, b_ode2)
